# Initial kernel scaffold; baseline (speedup 1.0000x reference)
import jax
import jax.numpy as jnp
from jax.experimental import pallas as pl


def kernel(conv1_w, conv1_b, conv2_w, conv2_b, fc1_w, fc1_b, fc2_w, fc2_b, x):
    raise NotImplementedError("write your pallas kernel here")


# speedup vs baseline: 3.0516x; 3.0516x over previous
import functools

import numpy as np

import jax
import jax.numpy as jnp
from jax.experimental import pallas as pl
from jax.experimental.pallas import tpu as pltpu

_K = 5


def _phase_split(x, pad_rows=3):
    b, c, h, w = x.shape
    hq, wq = h // 2, w // 2
    t = x.reshape(b, c, hq, 2, wq, 2)
    t = t.transpose(0, 3, 5, 1, 2, 4).reshape(b, 4, c, hq, wq)
    t = jnp.pad(t, ((0, 0), (0, 0), (0, 0), (0, pad_rows), (0, 0)))
    return t.reshape(b, 4, c, (hq + pad_rows) * wq)


@functools.lru_cache(maxsize=None)
def _tap_placement():
    t = np.zeros((4, _K * _K, 36), np.float32)
    for po in range(4):
        pa, pb = po // 2, po % 2
        for kh in range(_K):
            for kw in range(_K):
                t[po, kh * _K + kw, (pa + kh) * 6 + (pb + kw)] = 1.0
    return t


def _stack_weights(w_taps):
    cout, cin = w_taps.shape[1], w_taps.shape[2]
    t = jnp.asarray(_tap_placement())
    ws = jnp.einsum("toi,ptr->pori", w_taps, t)
    return ws.reshape(4 * cout, 36 * cin)


def _conv_body(xq_ref, w_ref, b_ref, o_ref, *, cout, wq, nd, img):
    w = w_ref[...]
    b = b_ref[...]
    for i in range(img):
        parts = []
        for r in range(6):
            for c in range(6):
                ph = (r % 2) * 2 + (c % 2)
                off = (r // 2) * wq + (c // 2)
                parts.append(xq_ref[i, ph, :, off:off + nd])
        xa = jnp.concatenate(parts, axis=0)
        acc = jnp.dot(w, xa, preferred_element_type=jnp.float32)
        pooled = jnp.maximum(
            jnp.maximum(acc[:cout], acc[cout:2 * cout]),
            jnp.maximum(acc[2 * cout:3 * cout], acc[3 * cout:]))
        o_ref[i] = jnp.maximum(pooled + b, 0.0)


def _fc_body(x_ref, w1_ref, b1_ref, w2_ref, b2_ref, o_ref):
    h = jnp.dot(x_ref[...], w1_ref[...], preferred_element_type=jnp.float32)
    h = jnp.maximum(h + b1_ref[...], 0.0)
    z = jnp.dot(h, w2_ref[...], preferred_element_type=jnp.float32)
    z = z + b2_ref[...]
    z = z - jnp.max(z, axis=-1, keepdims=True)
    o_ref[...] = z - jnp.log(jnp.sum(jnp.exp(z), axis=-1, keepdims=True))


def _conv_stage(xq, w_stack, bias, *, cout, cin, wq, nd, img):
    b = xq.shape[0]
    lsrc = xq.shape[-1]
    body = functools.partial(_conv_body, cout=cout, wq=wq, nd=nd, img=img)
    return pl.pallas_call(
        body,
        out_shape=jax.ShapeDtypeStruct((b, cout, nd), jnp.float32),
        grid=(b // img,),
        in_specs=[
            pl.BlockSpec((img, 4, cin, lsrc), lambda i: (i, 0, 0, 0)),
            pl.BlockSpec((4 * cout, 36 * cin), lambda i: (0, 0)),
            pl.BlockSpec((cout, 1), lambda i: (0, 0)),
        ],
        out_specs=pl.BlockSpec((img, cout, nd), lambda i: (i, 0, 0)),
        compiler_params=pltpu.CompilerParams(
            dimension_semantics=("parallel",)),
    )(xq, w_stack, bias)


def _fc_stage(x, w1, b1, w2, b2, bt):
    b, d = x.shape
    h1 = w1.shape[1]
    h2 = w2.shape[1]
    return pl.pallas_call(
        _fc_body,
        out_shape=jax.ShapeDtypeStruct((b, h2), jnp.float32),
        grid=(b // bt,),
        in_specs=[
            pl.BlockSpec((bt, d), lambda i: (i, 0)),
            pl.BlockSpec((d, h1), lambda i: (0, 0)),
            pl.BlockSpec((1, h1), lambda i: (0, 0)),
            pl.BlockSpec((h1, h2), lambda i: (0, 0)),
            pl.BlockSpec((1, h2), lambda i: (0, 0)),
        ],
        out_specs=pl.BlockSpec((bt, h2), lambda i: (i, 0)),
        compiler_params=pltpu.CompilerParams(
            dimension_semantics=("parallel",)),
    )(x, w1, b1, w2, b2)


def kernel(conv1_w, conv1_b, conv2_w, conv2_b, fc1_w, fc1_b, fc2_w, fc2_b, x):
    bsz = x.shape[0]
    img1 = 8 if bsz % 8 == 0 else 1
    img2 = 8 if bsz % 8 == 0 else 1
    bt = 128 if bsz % 128 == 0 else bsz

    w1s = _stack_weights(conv1_w)
    w2s = _stack_weights(conv2_w)

    xq1 = _phase_split(x)
    y1 = _conv_stage(xq1, w1s, conv1_b,
                     cout=10, cin=3, wq=42, nd=40 * 42, img=img1)
    y1 = y1.reshape(bsz, 10, 40, 42)

    xq2 = _phase_split(y1)
    y2 = _conv_stage(xq2, w2s, conv2_b,
                     cout=20, cin=10, wq=21, nd=18 * 21, img=img2)

    y2 = y2.reshape(bsz, 20 * 18 * 21)
    return _fc_stage(y2, fc1_w, fc1_b, fc2_w, fc2_b, bt)


# speedup vs baseline: 4.0745x; 1.3352x over previous
import functools

import numpy as np

import jax
import jax.numpy as jnp
from jax.experimental import pallas as pl
from jax.experimental.pallas import tpu as pltpu

_K = 5
_CO1 = 16
_CO2 = 24
_WQ = 21
_ND1 = 21 * _WQ
_ND2 = 18 * _WQ
_L16 = 23 * _WQ


def _phase16_split(x):
    b, c, h, w = x.shape
    hq, wq = h // 4, w // 4
    t = x.reshape(b, c, hq, 4, wq, 4).transpose(0, 3, 5, 1, 2, 4)
    t = t.reshape(b, 16, c, hq, wq)
    t = jnp.pad(t, ((0, 0), (0, 0), (0, 0), (0, 2), (0, 0)))
    return t.reshape(b, 16, c, (hq + 2) * wq)


@functools.lru_cache(maxsize=None)
def _tap_placement():
    t = np.zeros((4, _K * _K, 36), np.float32)
    for po in range(4):
        pa, pb = po // 2, po % 2
        for kh in range(_K):
            for kw in range(_K):
                t[po, kh * _K + kw, (pa + kh) * 6 + (pb + kw)] = 1.0
    return t


def _stack_weights(w_taps, cout_pad, cin_pad):
    cout, cin = w_taps.shape[1], w_taps.shape[2]
    t = jnp.asarray(_tap_placement())
    ws = jnp.einsum("toi,ptr->pori", w_taps, t)
    ws = jnp.pad(ws, ((0, 0), (0, cout_pad - cout), (0, 0), (0, cin_pad - cin)))
    return ws.reshape(4 * cout_pad, 36 * cin_pad)


def _pad_rows(v, n):
    return jnp.pad(v, ((0, n - v.shape[0]), (0, 0)))


def _conv_tower_body(xq_ref, w1_ref, b1_ref, w2_ref, b2_ref, o_ref, *,
                     cin, img):
    w1 = w1_ref[...]
    b1 = b1_ref[...]
    w2 = w2_ref[...]
    b2 = b2_ref[...]
    for i in range(img):
        y1 = []
        for p in (0, 1):
            for q in (0, 1):
                parts = []
                for s in range(6):
                    for t in range(6):
                        ph = ((2 * p + s) % 4) * 4 + (2 * q + t) % 4
                        off = ((2 * p + s) // 4) * _WQ + (2 * q + t) // 4
                        parts.append(xq_ref[i, ph, :, off:off + _ND1])
                xa = jnp.concatenate(parts, axis=0)
                acc = jnp.dot(w1, xa, preferred_element_type=jnp.float32)
                pooled = jnp.maximum(
                    jnp.maximum(acc[:_CO1], acc[_CO1:2 * _CO1]),
                    jnp.maximum(acc[2 * _CO1:3 * _CO1], acc[3 * _CO1:]))
                y1.append(jnp.maximum(pooled + b1, 0.0))
        parts = []
        for r in range(6):
            for c in range(6):
                ph = (r % 2) * 2 + (c % 2)
                off = (r // 2) * _WQ + (c // 2)
                parts.append(y1[ph][:, off:off + _ND2])
        xa = jnp.concatenate(parts, axis=0)
        acc = jnp.dot(w2, xa, preferred_element_type=jnp.float32)
        pooled = jnp.maximum(
            jnp.maximum(acc[:_CO2], acc[_CO2:2 * _CO2]),
            jnp.maximum(acc[2 * _CO2:3 * _CO2], acc[3 * _CO2:]))
        o_ref[i] = jnp.maximum(pooled + b2, 0.0)


def _fc_body(x_ref, w1_ref, b1_ref, w2_ref, b2_ref, o_ref):
    h = jnp.dot(x_ref[...], w1_ref[...], preferred_element_type=jnp.float32)
    h = jnp.maximum(h + b1_ref[...], 0.0)
    z = jnp.dot(h, w2_ref[...], preferred_element_type=jnp.float32)
    z = z + b2_ref[...]
    z = z - jnp.max(z, axis=-1, keepdims=True)
    o_ref[...] = z - jnp.log(jnp.sum(jnp.exp(z), axis=-1, keepdims=True))


def _conv_tower(xq, w1s, b1, w2s, b2, *, cin, img):
    b = xq.shape[0]
    body = functools.partial(_conv_tower_body, cin=cin, img=img)
    return pl.pallas_call(
        body,
        out_shape=jax.ShapeDtypeStruct((b, _CO2, _ND2), jnp.float32),
        grid=(b // img,),
        in_specs=[
            pl.BlockSpec((img, 16, cin, _L16), lambda i: (i, 0, 0, 0)),
            pl.BlockSpec((4 * _CO1, 36 * cin), lambda i: (0, 0)),
            pl.BlockSpec((_CO1, 1), lambda i: (0, 0)),
            pl.BlockSpec((4 * _CO2, 36 * _CO1), lambda i: (0, 0)),
            pl.BlockSpec((_CO2, 1), lambda i: (0, 0)),
        ],
        out_specs=pl.BlockSpec((img, _CO2, _ND2), lambda i: (i, 0, 0)),
        compiler_params=pltpu.CompilerParams(
            dimension_semantics=("parallel",)),
    )(xq, w1s, b1, w2s, b2)


def _fc_stage(x, w1, b1, w2, b2, bt):
    b, d = x.shape
    h1 = w1.shape[1]
    h2 = w2.shape[1]
    return pl.pallas_call(
        _fc_body,
        out_shape=jax.ShapeDtypeStruct((b, h2), jnp.float32),
        grid=(b // bt,),
        in_specs=[
            pl.BlockSpec((bt, d), lambda i: (i, 0)),
            pl.BlockSpec((d, h1), lambda i: (0, 0)),
            pl.BlockSpec((1, h1), lambda i: (0, 0)),
            pl.BlockSpec((h1, h2), lambda i: (0, 0)),
            pl.BlockSpec((1, h2), lambda i: (0, 0)),
        ],
        out_specs=pl.BlockSpec((bt, h2), lambda i: (i, 0)),
        compiler_params=pltpu.CompilerParams(
            dimension_semantics=("parallel",)),
    )(x, w1, b1, w2, b2)


def kernel(conv1_w, conv1_b, conv2_w, conv2_b, fc1_w, fc1_b, fc2_w, fc2_b, x):
    bsz, cin = x.shape[0], x.shape[1]
    img = 8 if bsz % 8 == 0 else 1
    bt = 128 if bsz % 128 == 0 else bsz

    w1s = _stack_weights(conv1_w, _CO1, cin)
    w2s = _stack_weights(conv2_w, _CO2, _CO1)
    b1 = _pad_rows(conv1_b, _CO1)
    b2 = _pad_rows(conv2_b, _CO2)
    f1 = fc1_w.reshape(conv2_w.shape[1], _ND2, fc1_w.shape[1])
    f1 = jnp.pad(f1, ((0, _CO2 - conv2_w.shape[1]), (0, 0), (0, 0)))
    f1 = f1.reshape(_CO2 * _ND2, fc1_w.shape[1])

    xq = _phase16_split(x)
    y2 = _conv_tower(xq, w1s, b1, w2s, b2, cin=cin, img=img)
    y2 = y2.reshape(bsz, _CO2 * _ND2)
    return _fc_stage(y2, f1, fc1_b, fc2_w, fc2_b, bt)
